# baseline (speedup 1.0000x reference)
import functools

import jax
import jax.numpy as jnp
from jax import lax
from jax.experimental import pallas as pl
from jax.experimental.pallas import tpu as pltpu
from jax.experimental.pallas import tpu_sc as plsc

_NC = 2
_NS = 16
_NW = _NC * _NS
_CHUNK = 128
_NBUF = 2
_IB = 16


@functools.lru_cache(maxsize=None)
def _sc_agg(n_acc, d, cpw):
    rpt = n_acc // _NS
    mesh = plsc.VectorSubcoreMesh(core_axis_name="c", subcore_axis_name="s")

    @functools.partial(
        pl.kernel,
        mesh=mesh,
        out_type=[
            jax.ShapeDtypeStruct((_NC, n_acc, d), jnp.float32),
            jax.ShapeDtypeStruct((_NC, n_acc), jnp.float32),
        ],
        scratch_types=[
            pltpu.VMEM((2, _IB, _CHUNK), jnp.int32),
            pltpu.VMEM((2, _IB, _CHUNK), jnp.int32),
            pltpu.VMEM((_NBUF, _CHUNK, d), jnp.float32),
            pltpu.VMEM((_CHUNK,), jnp.float32),
            pltpu.VMEM_SHARED((n_acc, d), jnp.float32),
            pltpu.VMEM_SHARED((n_acc,), jnp.float32),
        ] + [pltpu.SemaphoreType.DMA] * (_NBUF + 3),
    )
    def agg(table, src3, dst3, zrows, zdeg, sums, degs,
            src_v, dst_v, rows_v, ones_v, acc, deg, *sems):
        isrc, idst, dsem = sems[_NBUF], sems[_NBUF + 1], sems[_NBUF + 2]
        c = lax.axis_index("c")
        s = lax.axis_index("s")
        wid = c * _NS + s
        row0 = s * rpt
        nblk = cpw // _IB
        pltpu.sync_copy(zrows.at[pl.ds(row0, rpt)], acc.at[pl.ds(row0, rpt)])
        pltpu.sync_copy(zdeg.at[pl.ds(row0, rpt)], deg.at[pl.ds(row0, rpt)])
        for i in range(_CHUNK // 16):
            ones_v[pl.ds(i * 16, 16)] = jnp.ones((16,), jnp.float32)
        pltpu.sync_copy(src3.at[wid, pl.ds(0, _IB)], src_v.at[0])
        pltpu.sync_copy(dst3.at[wid, pl.ds(0, _IB)], dst_v.at[0])
        plsc.subcore_barrier()
        for b in range(_NBUF):
            pltpu.async_copy(table.at[src_v.at[0, b]], rows_v.at[b], sems[b])

        def blk_body(k, carry):
            par = lax.rem(k, 2)
            nxt_par = 1 - par

            @pl.when(k > 0)
            def _():
                pltpu.make_async_copy(ones_v, deg.at[dst_v.at[par, 0]],
                                      dsem).wait()

            @pl.when(k + 1 < nblk)
            def _():
                off = (k + 1) * _IB
                pltpu.async_copy(src3.at[wid, pl.ds(off, _IB)],
                                 src_v.at[nxt_par], isrc)
                pltpu.async_copy(dst3.at[wid, pl.ds(off, _IB)],
                                 dst_v.at[nxt_par], idst)

            for i in range(_IB):
                slot = i % _NBUF
                if i + _NBUF == _IB:
                    @pl.when(k + 1 < nblk)
                    def _():
                        pltpu.make_async_copy(
                            src3.at[wid, pl.ds(0, _IB)], src_v.at[nxt_par],
                            isrc).wait()
                        pltpu.make_async_copy(
                            dst3.at[wid, pl.ds(0, _IB)], dst_v.at[nxt_par],
                            idst).wait()
                pltpu.make_async_copy(
                    table.at[src_v.at[par, i]], rows_v.at[slot],
                    sems[slot]).wait()
                pltpu.sync_copy(rows_v.at[slot], acc.at[dst_v.at[par, i]],
                                add=True)
                if i + _NBUF < _IB:
                    pltpu.async_copy(table.at[src_v.at[par, i + _NBUF]],
                                     rows_v.at[slot], sems[slot])
                else:
                    @pl.when(k + 1 < nblk)
                    def _():
                        pltpu.async_copy(
                            table.at[src_v.at[nxt_par, i + _NBUF - _IB]],
                            rows_v.at[slot], sems[slot])
                if i > 0:
                    pltpu.make_async_copy(
                        ones_v, deg.at[dst_v.at[par, i]], dsem).wait()
                pltpu.async_copy(ones_v, deg.at[dst_v.at[par, i]], dsem,
                                 add=True)
            return carry

        lax.fori_loop(0, nblk, blk_body, 0)
        pltpu.make_async_copy(ones_v, deg.at[dst_v.at[0, 0]], dsem).wait()
        plsc.subcore_barrier()
        pltpu.sync_copy(acc.at[pl.ds(row0, rpt)], sums.at[c, pl.ds(row0, rpt)])
        pltpu.sync_copy(deg.at[pl.ds(row0, rpt)], degs.at[c, pl.ds(row0, rpt)])

    return agg


def _tc_self_body(x_ref, ws_ref, b_ref, z_ref):
    z_ref[...] = jnp.dot(x_ref[...], ws_ref[...],
                         preferred_element_type=jnp.float32) + b_ref[...]


def _tc_self(xd, ws, b, n, blk):
    d, d2 = ws.shape
    return pl.pallas_call(
        _tc_self_body,
        grid=(n // blk,),
        in_specs=[
            pl.BlockSpec((blk, d), lambda i: (i, 0)),
            pl.BlockSpec((d, d2), lambda i: (0, 0)),
            pl.BlockSpec((1, d2), lambda i: (0, 0)),
        ],
        out_specs=pl.BlockSpec((blk, d2), lambda i: (i, 0)),
        out_shape=jax.ShapeDtypeStruct((n, d2), jnp.float32),
    )(xd, ws, b)


def _tc1_body(p_ref, deg_ref, zs_ref, wn_ref, h_ref):
    dsum = deg_ref[:, 0:1] + deg_ref[:, 1:2]
    inv = 1.0 / jnp.maximum(dsum, 1.0)
    hn = (p_ref[0] + p_ref[1]) * inv
    z = jnp.dot(hn, wn_ref[...], preferred_element_type=jnp.float32)
    h_ref[...] = jnp.maximum(z + zs_ref[...], 0.0)


def _tc_layer1(p, degt, zs, wn):
    n, d = p.shape[1], p.shape[2]
    blk = 2048
    return pl.pallas_call(
        _tc1_body,
        grid=(n // blk,),
        in_specs=[
            pl.BlockSpec((_NC, blk, d), lambda i: (0, i, 0)),
            pl.BlockSpec((blk, _NC), lambda i: (i, 0)),
            pl.BlockSpec((blk, d), lambda i: (i, 0)),
            pl.BlockSpec((d, d), lambda i: (0, 0)),
        ],
        out_specs=pl.BlockSpec((blk, d), lambda i: (i, 0)),
        out_shape=jax.ShapeDtypeStruct((n, d), jnp.float32),
    )(p, degt, zs, wn)


def _tc2_body(p_ref, deg_ref, zs_ref, wn_ref, out_ref):
    dsum = deg_ref[:, 0:1] + deg_ref[:, 1:2]
    inv = 1.0 / jnp.maximum(dsum, 1.0)
    hn = (p_ref[0] + p_ref[1]) * inv
    z = jnp.dot(hn, wn_ref[...], preferred_element_type=jnp.float32)
    out_ref[...] = z + zs_ref[...]


def _tc_layer2(p, degt, zs, wn):
    n, d = p.shape[1], p.shape[2]
    d2 = wn.shape[1]
    return pl.pallas_call(
        _tc2_body,
        grid=(1,),
        in_specs=[
            pl.BlockSpec((_NC, n, d), lambda i: (0, 0, 0)),
            pl.BlockSpec((n, _NC), lambda i: (0, 0)),
            pl.BlockSpec((n, d2), lambda i: (0, 0)),
            pl.BlockSpec((d, d2), lambda i: (0, 0)),
        ],
        out_specs=pl.BlockSpec((n, d2), lambda i: (0, 0)),
        out_shape=jax.ShapeDtypeStruct((n, d2), jnp.float32),
    )(p, degt, zs, wn)


def _pad_edges(src, dst, n_src, n_dst, n_acc):
    e = src.shape[0]
    cpw = -(-e // (_NW * _CHUNK))
    cpw = -(-cpw // _IB) * _IB
    ep = _NW * _CHUNK * cpw
    ar = jnp.arange(ep - e, dtype=src.dtype)
    src_p = jnp.concatenate([src, ar % n_src])
    dst_p = jnp.concatenate([dst, n_dst + ar % (n_acc - n_dst)])
    return (src_p.reshape(_NW, cpw, _CHUNK),
            dst_p.reshape(_NW, cpw, _CHUNK), cpw)


def kernel(x, src1, dst1, src2, dst2, num_dst1, num_dst2,
           W_neigh1, W_self1, b1, W_neigh2, W_self2, b2):
    n1, n2 = 10000, 2000
    n1p, n2p = 10240, 2048
    d_in = x.shape[1]
    d_h = W_neigh1.shape[1]
    d_out = W_neigh2.shape[1]
    dst1 = dst1 + (jnp.asarray(num_dst1, dst1.dtype) - n1)
    dst2 = dst2 + (jnp.asarray(num_dst2, dst2.dtype) - n2)

    src1p, dst1p, cpw1 = _pad_edges(src1, dst1, x.shape[0], n1, n1p)
    src2p, dst2p, cpw2 = _pad_edges(src2, dst2, n1, n2, n2p)

    z1r = jnp.zeros((n1p, d_in), jnp.float32)
    z1d = jnp.zeros((n1p,), jnp.float32)
    sums1, degs1 = _sc_agg(n1p, d_in, cpw1)(x, src1p, dst1p, z1r, z1d)
    zs1 = _tc_self(x, W_self1, b1.reshape(1, d_h), n1p, 1024)

    h = _tc_layer1(sums1, degs1.T, zs1, W_neigh1)

    z2r = jnp.zeros((n2p, d_h), jnp.float32)
    z2d = jnp.zeros((n2p,), jnp.float32)
    sums2, degs2 = _sc_agg(n2p, d_h, cpw2)(h, src2p, dst2p, z2r, z2d)
    zs2 = _tc_self(h, W_self2, b2.reshape(1, d_out), n2p, 2048)

    out = _tc_layer2(sums2, degs2.T, zs2, W_neigh2)
    return out[:n2]


# speedup vs baseline: 1.0114x; 1.0114x over previous
import functools

import jax
import jax.numpy as jnp
from jax import lax
from jax.experimental import pallas as pl
from jax.experimental.pallas import tpu as pltpu
from jax.experimental.pallas import tpu_sc as plsc

_NC = 2
_NS = 16
_NW = _NC * _NS
_CHUNK = 128
_NBUF = 2
_IB = 16


@functools.lru_cache(maxsize=None)
def _sc_agg(n_acc, d, cpw):
    rpt = n_acc // _NS
    mesh = plsc.VectorSubcoreMesh(core_axis_name="c", subcore_axis_name="s")

    @functools.partial(
        pl.kernel,
        mesh=mesh,
        out_type=[
            jax.ShapeDtypeStruct((_NC, n_acc, d), jnp.float32),
            jax.ShapeDtypeStruct((_NC, n_acc), jnp.float32),
        ],
        scratch_types=[
            pltpu.VMEM((2, _IB, _CHUNK), jnp.int32),
            pltpu.VMEM((2, _IB, _CHUNK), jnp.int32),
            pltpu.VMEM((_NBUF, _CHUNK, d), jnp.float32),
            pltpu.VMEM((_CHUNK,), jnp.float32),
            pltpu.VMEM_SHARED((n_acc, d), jnp.float32),
            pltpu.VMEM_SHARED((n_acc,), jnp.float32),
        ] + [pltpu.SemaphoreType.DMA] * (_NBUF + 2),
    )
    def agg(table, src3, dst3, zrows, zdeg, sums, degs,
            src_v, dst_v, rows_v, ones_v, acc, deg, *sems):
        isrc, idst = sems[_NBUF], sems[_NBUF + 1]
        c = lax.axis_index("c")
        s = lax.axis_index("s")
        wid = c * _NS + s
        row0 = s * rpt
        nblk = cpw // _IB
        pltpu.sync_copy(zrows.at[pl.ds(row0, rpt)], acc.at[pl.ds(row0, rpt)])
        pltpu.sync_copy(zdeg.at[pl.ds(row0, rpt)], deg.at[pl.ds(row0, rpt)])
        for i in range(_CHUNK // 16):
            ones_v[pl.ds(i * 16, 16)] = jnp.ones((16,), jnp.float32)
        pltpu.sync_copy(src3.at[wid, pl.ds(0, _IB)], src_v.at[0])
        pltpu.sync_copy(dst3.at[wid, pl.ds(0, _IB)], dst_v.at[0])
        plsc.subcore_barrier()
        for b in range(_NBUF):
            pltpu.async_copy(table.at[src_v.at[0, b]], rows_v.at[b], sems[b])

        def blk_body(k, carry):
            par = lax.rem(k, 2)
            nxt_par = 1 - par

            @pl.when(k + 1 < nblk)
            def _():
                off = (k + 1) * _IB
                pltpu.async_copy(src3.at[wid, pl.ds(off, _IB)],
                                 src_v.at[nxt_par], isrc)
                pltpu.async_copy(dst3.at[wid, pl.ds(off, _IB)],
                                 dst_v.at[nxt_par], idst)

            for i in range(_IB):
                slot = i % _NBUF
                if i + _NBUF == _IB:
                    @pl.when(k + 1 < nblk)
                    def _():
                        pltpu.make_async_copy(
                            src3.at[wid, pl.ds(0, _IB)], src_v.at[nxt_par],
                            isrc).wait()
                        pltpu.make_async_copy(
                            dst3.at[wid, pl.ds(0, _IB)], dst_v.at[nxt_par],
                            idst).wait()
                pltpu.make_async_copy(
                    table.at[src_v.at[par, i]], rows_v.at[slot],
                    sems[slot]).wait()
                pltpu.sync_copy(rows_v.at[slot], acc.at[dst_v.at[par, i]],
                                add=True)
                if i + _NBUF < _IB:
                    pltpu.async_copy(table.at[src_v.at[par, i + _NBUF]],
                                     rows_v.at[slot], sems[slot])
                else:
                    @pl.when(k + 1 < nblk)
                    def _():
                        pltpu.async_copy(
                            table.at[src_v.at[nxt_par, i + _NBUF - _IB]],
                            rows_v.at[slot], sems[slot])
                pltpu.sync_copy(ones_v, deg.at[dst_v.at[par, i]], add=True)
            return carry

        lax.fori_loop(0, nblk, blk_body, 0)
        plsc.subcore_barrier()
        pltpu.sync_copy(acc.at[pl.ds(row0, rpt)], sums.at[c, pl.ds(row0, rpt)])
        pltpu.sync_copy(deg.at[pl.ds(row0, rpt)], degs.at[c, pl.ds(row0, rpt)])

    return agg


def _tc_self_body(x_ref, ws_ref, b_ref, z_ref):
    z_ref[...] = jnp.dot(x_ref[...], ws_ref[...],
                         preferred_element_type=jnp.float32) + b_ref[...]


def _tc_self(xd, ws, b, n, blk):
    d, d2 = ws.shape
    return pl.pallas_call(
        _tc_self_body,
        grid=(n // blk,),
        in_specs=[
            pl.BlockSpec((blk, d), lambda i: (i, 0)),
            pl.BlockSpec((d, d2), lambda i: (0, 0)),
            pl.BlockSpec((1, d2), lambda i: (0, 0)),
        ],
        out_specs=pl.BlockSpec((blk, d2), lambda i: (i, 0)),
        out_shape=jax.ShapeDtypeStruct((n, d2), jnp.float32),
    )(xd, ws, b)


def _tc1_body(p_ref, deg_ref, zs_ref, wn_ref, h_ref):
    dsum = deg_ref[:, 0:1] + deg_ref[:, 1:2]
    inv = 1.0 / jnp.maximum(dsum, 1.0)
    hn = (p_ref[0] + p_ref[1]) * inv
    z = jnp.dot(hn, wn_ref[...], preferred_element_type=jnp.float32)
    h_ref[...] = jnp.maximum(z + zs_ref[...], 0.0)


def _tc_layer1(p, degt, zs, wn):
    n, d = p.shape[1], p.shape[2]
    blk = 2048
    return pl.pallas_call(
        _tc1_body,
        grid=(n // blk,),
        in_specs=[
            pl.BlockSpec((_NC, blk, d), lambda i: (0, i, 0)),
            pl.BlockSpec((blk, _NC), lambda i: (i, 0)),
            pl.BlockSpec((blk, d), lambda i: (i, 0)),
            pl.BlockSpec((d, d), lambda i: (0, 0)),
        ],
        out_specs=pl.BlockSpec((blk, d), lambda i: (i, 0)),
        out_shape=jax.ShapeDtypeStruct((n, d), jnp.float32),
    )(p, degt, zs, wn)


def _tc2_body(p_ref, deg_ref, zs_ref, wn_ref, out_ref):
    dsum = deg_ref[:, 0:1] + deg_ref[:, 1:2]
    inv = 1.0 / jnp.maximum(dsum, 1.0)
    hn = (p_ref[0] + p_ref[1]) * inv
    z = jnp.dot(hn, wn_ref[...], preferred_element_type=jnp.float32)
    out_ref[...] = z + zs_ref[...]


def _tc_layer2(p, degt, zs, wn):
    n, d = p.shape[1], p.shape[2]
    d2 = wn.shape[1]
    return pl.pallas_call(
        _tc2_body,
        grid=(1,),
        in_specs=[
            pl.BlockSpec((_NC, n, d), lambda i: (0, 0, 0)),
            pl.BlockSpec((n, _NC), lambda i: (0, 0)),
            pl.BlockSpec((n, d2), lambda i: (0, 0)),
            pl.BlockSpec((d, d2), lambda i: (0, 0)),
        ],
        out_specs=pl.BlockSpec((n, d2), lambda i: (0, 0)),
        out_shape=jax.ShapeDtypeStruct((n, d2), jnp.float32),
    )(p, degt, zs, wn)


def _pad_edges(src, dst, n_src, n_dst, n_acc):
    e = src.shape[0]
    cpw = -(-e // (_NW * _CHUNK))
    cpw = -(-cpw // _IB) * _IB
    ep = _NW * _CHUNK * cpw
    ar = jnp.arange(ep - e, dtype=src.dtype)
    src_p = jnp.concatenate([src, ar % n_src])
    dst_p = jnp.concatenate([dst, n_dst + ar % (n_acc - n_dst)])
    return (src_p.reshape(_NW, cpw, _CHUNK),
            dst_p.reshape(_NW, cpw, _CHUNK), cpw)


def kernel(x, src1, dst1, src2, dst2, num_dst1, num_dst2,
           W_neigh1, W_self1, b1, W_neigh2, W_self2, b2):
    n1, n2 = 10000, 2000
    n1p, n2p = 10240, 2048
    d_in = x.shape[1]
    d_h = W_neigh1.shape[1]
    d_out = W_neigh2.shape[1]
    dst1 = dst1 + (jnp.asarray(num_dst1, dst1.dtype) - n1)
    dst2 = dst2 + (jnp.asarray(num_dst2, dst2.dtype) - n2)

    src1p, dst1p, cpw1 = _pad_edges(src1, dst1, x.shape[0], n1, n1p)
    src2p, dst2p, cpw2 = _pad_edges(src2, dst2, n1, n2, n2p)

    z1r = jnp.zeros((n1p, d_in), jnp.float32)
    z1d = jnp.zeros((n1p,), jnp.float32)
    sums1, degs1 = _sc_agg(n1p, d_in, cpw1)(x, src1p, dst1p, z1r, z1d)
    zs1 = _tc_self(x, W_self1, b1.reshape(1, d_h), n1p, 1024)

    h = _tc_layer1(sums1, degs1.T, zs1, W_neigh1)

    z2r = jnp.zeros((n2p, d_h), jnp.float32)
    z2d = jnp.zeros((n2p,), jnp.float32)
    sums2, degs2 = _sc_agg(n2p, d_h, cpw2)(h, src2p, dst2p, z2r, z2d)
    zs2 = _tc_self(h, W_self2, b2.reshape(1, d_out), n2p, 2048)

    out = _tc_layer2(sums2, degs2.T, zs2, W_neigh2)
    return out[:n2]
